# baseline (speedup 1.0000x reference)
import functools

import jax
import jax.numpy as jnp
from jax import lax
from jax.experimental import pallas as pl
from jax.experimental.pallas import tpu as pltpu
from jax.experimental.pallas import tpu_sc as plsc

_NHARM = 30
_HID = 64
_GRES = 32
_VOL = 3.0
_VOX = 2.0 * _VOL / _GRES
_R = 4096
_P = 64
_NPT = _R * _P

_NW = 32
_PPW = _NPT // _NW
_NVEC = _PPW // 16


def _sc_mask_body(pt3_hbm, grid_hbm, out_hbm, pt_v, grid_v, out_v):
    wid = lax.axis_index("s") * 2 + lax.axis_index("c")
    base = wid * _PPW
    pltpu.sync_copy(pt3_hbm.at[:, pl.ds(base, _PPW)], pt_v)
    pltpu.sync_copy(grid_hbm, grid_v)

    def step(k, carry):
        lin = None
        for c, mulc in ((0, _GRES * _GRES), (1, _GRES), (2, 1)):
            v = ((pt_v[c, pl.ds(k * 16, 16)] + _VOL) / _VOX).astype(jnp.int32)
            v = jnp.minimum(jnp.maximum(v, 0), _GRES - 1)
            lin = v * mulc if lin is None else lin + v * mulc
        out_v[pl.ds(k * 16, 16)] = plsc.load_gather(grid_v, [lin])
        return carry

    lax.fori_loop(0, _NVEC, step, 0)
    pltpu.sync_copy(out_v, out_hbm.at[pl.ds(base, _PPW)])


@functools.cache
def _make_sc_mask():
    return functools.partial(
        pl.kernel,
        mesh=plsc.VectorSubcoreMesh(core_axis_name="c", subcore_axis_name="s"),
        out_type=jax.ShapeDtypeStruct((_NPT,), jnp.float32),
        compiler_params=pltpu.CompilerParams(needs_layout_passes=False),
        scratch_types=[
            pltpu.VMEM((3, _PPW), jnp.float32),
            pltpu.VMEM((_GRES ** 3,), jnp.float32),
            pltpu.VMEM((_PPW,), jnp.float32),
        ],
    )(_sc_mask_body)


_B = 64


def _sp10(x):
    z = 10.0 * x
    return (jnp.maximum(z, 0.0) + jnp.log1p(jnp.exp(-jnp.abs(z)))) * 0.1


def _tc_body(p0_ref, p1_ref, p2_ref, dn_ref, m_ref, fr_ref,
             w1s_ref, w1c_ref, b1_ref, w2_ref, b2_ref,
             wc1h_ref, wc1ds_ref, wc1dc_ref, bc1_ref,
             wc2_ref, bc2_ref, wd_ref, bd_ref,
             dens_ref, col_ref):
    n = _B * _P
    f90 = fr_ref[...]

    pts = (p0_ref[...], p1_ref[...], p2_ref[...])
    lane3 = lax.broadcasted_iota(jnp.int32, (_B, _P, 3 * _NHARM), 2)
    psel = jnp.where(lane3 < _NHARM, pts[0][:, :, None],
                     jnp.where(lane3 < 2 * _NHARM, pts[1][:, :, None],
                               pts[2][:, :, None]))
    ang = (psel * f90[0][None, None, :]).reshape(n, 3 * _NHARM)

    h = _sp10(jnp.dot(jnp.sin(ang).astype(jnp.bfloat16), w1s_ref[...],
                      preferred_element_type=jnp.float32)
              + jnp.dot(jnp.cos(ang).astype(jnp.bfloat16), w1c_ref[...],
                        preferred_element_type=jnp.float32)
              + b1_ref[...])
    h = _sp10(jnp.dot(h.astype(jnp.bfloat16), w2_ref[...],
                      preferred_element_type=jnp.float32)
              + b2_ref[...])

    h3 = h.reshape(_B, _P, _HID)
    rd = jnp.sum(h3 * wd_ref[...][0][None, None, :], axis=-1) + bd_ref[0, 0]
    dens = 1.0 - jnp.exp(-_sp10(rd))

    dn = dn_ref[...]
    dl = lax.broadcasted_iota(jnp.int32, (_B, 3 * _NHARM), 1)
    dsel = jnp.where(dl < _NHARM, dn[:, 0:1],
                     jnp.where(dl < 2 * _NHARM, dn[:, 1:2], dn[:, 2:3]))
    dang = dsel * f90
    dproj = (jnp.dot(jnp.sin(dang), wc1ds_ref[...],
                     preferred_element_type=jnp.float32)
             + jnp.dot(jnp.cos(dang), wc1dc_ref[...],
                       preferred_element_type=jnp.float32)
             + bc1_ref[...])

    c1 = jnp.dot(h.astype(jnp.bfloat16), wc1h_ref[...],
                 preferred_element_type=jnp.float32)
    c1 = _sp10(c1.reshape(_B, _P, _HID) + dproj[:, None, :]).reshape(n, _HID)
    cpre = jnp.dot(c1, wc2_ref[...], preferred_element_type=jnp.float32) \
        + bc2_ref[...]
    col = 1.0 / (1.0 + jnp.exp(-cpre))

    m2 = m_ref[...]
    dens_ref[...] = dens * m2
    col_ref[...] = col.reshape(_B, _P, 3) * m2[:, :, None]


def _const_spec(shape):
    return pl.BlockSpec(shape, lambda i: tuple(0 for _ in shape))


_TC_IN_SPECS = [
    pl.BlockSpec((_B, _P), lambda i: (i, 0)),
    pl.BlockSpec((_B, _P), lambda i: (i, 0)),
    pl.BlockSpec((_B, _P), lambda i: (i, 0)),
    pl.BlockSpec((_B, 3), lambda i: (i, 0)),
    pl.BlockSpec((_B, _P), lambda i: (i, 0)),
    _const_spec((1, 3 * _NHARM)),
    _const_spec((3 * _NHARM, _HID)),
    _const_spec((3 * _NHARM, _HID)),
    _const_spec((1, _HID)),
    _const_spec((_HID, _HID)),
    _const_spec((1, _HID)),
    _const_spec((_HID, _HID)),
    _const_spec((3 * _NHARM, _HID)),
    _const_spec((3 * _NHARM, _HID)),
    _const_spec((1, _HID)),
    _const_spec((_HID, 3)),
    _const_spec((1, 3)),
    _const_spec((1, _HID)),
    _const_spec((1, 1)),
]

_TC_OUT_SPECS = [
    pl.BlockSpec((_B, _P), lambda i: (i, 0)),
    pl.BlockSpec((_B, _P, 3), lambda i: (i, 0, 0)),
]

_TC_OUT_SHAPE = [
    jax.ShapeDtypeStruct((_R, _P), jnp.float32),
    jax.ShapeDtypeStruct((_R, _P, 3), jnp.float32),
]


def _tc_call(*args):
    return pl.pallas_call(
        _tc_body,
        grid=(_R // _B,),
        in_specs=_TC_IN_SPECS,
        out_specs=_TC_OUT_SPECS,
        out_shape=_TC_OUT_SHAPE,
    )(*args)


def kernel(origins, directions, lengths, grid, frequencies,
           W1, b1, W2, b2, Wc1, bc1, Wc2, bc2, Wd, bd):
    pts = origins[:, None, :] + directions[:, None, :] * lengths[..., None]
    dn = directions / jnp.maximum(
        jnp.linalg.norm(directions, axis=-1, keepdims=True), 1e-12)
    gridf = grid.reshape(-1).astype(jnp.float32)

    pt3 = pts.reshape(_NPT, 3).T
    mask2d = _make_sc_mask()(pt3, gridf).reshape(_R, _P)

    f90 = jnp.tile(frequencies, 3).reshape(1, 3 * _NHARM)
    dens2d, colors = _tc_call(
        pts[..., 0], pts[..., 1], pts[..., 2], dn, mask2d, f90,
        W1[:3 * _NHARM], W1[3 * _NHARM:], b1.reshape(1, _HID),
        W2, b2.reshape(1, _HID),
        Wc1[:_HID], Wc1[_HID:_HID + 3 * _NHARM], Wc1[_HID + 3 * _NHARM:],
        bc1.reshape(1, _HID),
        Wc2, bc2.reshape(1, 3),
        Wd.reshape(1, _HID), bd.reshape(1, 1))
    return dens2d.reshape(_R, _P, 1), colors


# speedup vs baseline: 1.1015x; 1.1015x over previous
import functools

import jax
import jax.numpy as jnp
from jax import lax
from jax.experimental import pallas as pl
from jax.experimental.pallas import tpu as pltpu
from jax.experimental.pallas import tpu_sc as plsc

_NHARM = 30
_HID = 64
_GRES = 32
_VOL = 3.0
_VOX = 2.0 * _VOL / _GRES
_R = 4096
_P = 64
_NPT = _R * _P

_NW = 32
_PPW = _NPT // _NW
_NVEC = _PPW // 16


def _sc_mask_body(pt3_hbm, grid_hbm, out_hbm, pt_v, grid_v, out_v):
    wid = lax.axis_index("s") * 2 + lax.axis_index("c")
    base = wid * _PPW
    pltpu.sync_copy(pt3_hbm.at[:, pl.ds(base, _PPW)], pt_v)
    pltpu.sync_copy(grid_hbm, grid_v)

    def step(k, carry):
        lin = None
        for c, mulc in ((0, _GRES * _GRES), (1, _GRES), (2, 1)):
            v = ((pt_v[c, pl.ds(k * 16, 16)] + _VOL) / _VOX).astype(jnp.int32)
            v = jnp.minimum(jnp.maximum(v, 0), _GRES - 1)
            lin = v * mulc if lin is None else lin + v * mulc
        out_v[pl.ds(k * 16, 16)] = plsc.load_gather(grid_v, [lin])
        return carry

    lax.fori_loop(0, _NVEC, step, 0)
    pltpu.sync_copy(out_v, out_hbm.at[pl.ds(base, _PPW)])


@functools.cache
def _make_sc_mask():
    return functools.partial(
        pl.kernel,
        mesh=plsc.VectorSubcoreMesh(core_axis_name="c", subcore_axis_name="s"),
        out_type=jax.ShapeDtypeStruct((_NPT,), jnp.float32),
        compiler_params=pltpu.CompilerParams(needs_layout_passes=False),
        scratch_types=[
            pltpu.VMEM((3, _PPW), jnp.float32),
            pltpu.VMEM((_GRES ** 3,), jnp.float32),
            pltpu.VMEM((_PPW,), jnp.float32),
        ],
    )(_sc_mask_body)


_B = 64


_LOG2E = 1.4426950408889634
_LN2 = 0.6931471805599453


def _sp10(x):
    z = 10.0 * x
    t = jnp.exp2(jnp.abs(z) * -_LOG2E)
    return (jnp.maximum(z, 0.0) + jnp.log2(1.0 + t) * _LN2) * 0.1


def _tc_body(p0_ref, p1_ref, p2_ref, dn_ref, m_ref, fr_ref,
             w1s_ref, w1c_ref, b1_ref, w2_ref, b2_ref,
             wc1h_ref, wc1ds_ref, wc1dc_ref, bc1_ref,
             wc2_ref, bc2_ref, wd_ref, bd_ref,
             dens_ref, col_ref):
    n = _B * _P
    f90 = fr_ref[...]

    pts = (p0_ref[...], p1_ref[...], p2_ref[...])
    lane3 = lax.broadcasted_iota(jnp.int32, (_B, _P, 3 * _NHARM), 2)
    psel = jnp.where(lane3 < _NHARM, pts[0][:, :, None],
                     jnp.where(lane3 < 2 * _NHARM, pts[1][:, :, None],
                               pts[2][:, :, None]))
    ang = (psel * f90[0][None, None, :]).reshape(n, 3 * _NHARM)

    h = _sp10(jnp.dot(jnp.sin(ang), w1s_ref[...],
                      preferred_element_type=jnp.float32)
              + jnp.dot(jnp.cos(ang), w1c_ref[...],
                        preferred_element_type=jnp.float32)
              + b1_ref[...])
    h = _sp10(jnp.dot(h, w2_ref[...], preferred_element_type=jnp.float32)
              + b2_ref[...])

    h3 = h.reshape(_B, _P, _HID)
    rd = jnp.sum(h3 * wd_ref[...][0][None, None, :], axis=-1) + bd_ref[0, 0]
    dens = 1.0 - jnp.exp(-_sp10(rd))

    dn = dn_ref[...]
    dl = lax.broadcasted_iota(jnp.int32, (_B, 3 * _NHARM), 1)
    dsel = jnp.where(dl < _NHARM, dn[:, 0:1],
                     jnp.where(dl < 2 * _NHARM, dn[:, 1:2], dn[:, 2:3]))
    dang = dsel * f90
    dproj = (jnp.dot(jnp.sin(dang), wc1ds_ref[...],
                     preferred_element_type=jnp.float32)
             + jnp.dot(jnp.cos(dang), wc1dc_ref[...],
                       preferred_element_type=jnp.float32)
             + bc1_ref[...])

    c1 = jnp.dot(h, wc1h_ref[...], preferred_element_type=jnp.float32)
    c1 = _sp10(c1.reshape(_B, _P, _HID) + dproj[:, None, :]).reshape(n, _HID)
    cpre = jnp.dot(c1, wc2_ref[...], preferred_element_type=jnp.float32) \
        + bc2_ref[...]
    col = 1.0 / (1.0 + jnp.exp(-cpre))

    m2 = m_ref[...]
    dens_ref[...] = dens * m2
    col_ref[...] = col.reshape(_B, _P, 3) * m2[:, :, None]


def _const_spec(shape):
    return pl.BlockSpec(shape, lambda i: tuple(0 for _ in shape))


_TC_IN_SPECS = [
    pl.BlockSpec((_B, _P), lambda i: (i, 0)),
    pl.BlockSpec((_B, _P), lambda i: (i, 0)),
    pl.BlockSpec((_B, _P), lambda i: (i, 0)),
    pl.BlockSpec((_B, 3), lambda i: (i, 0)),
    pl.BlockSpec((_B, _P), lambda i: (i, 0)),
    _const_spec((1, 3 * _NHARM)),
    _const_spec((3 * _NHARM, _HID)),
    _const_spec((3 * _NHARM, _HID)),
    _const_spec((1, _HID)),
    _const_spec((_HID, _HID)),
    _const_spec((1, _HID)),
    _const_spec((_HID, _HID)),
    _const_spec((3 * _NHARM, _HID)),
    _const_spec((3 * _NHARM, _HID)),
    _const_spec((1, _HID)),
    _const_spec((_HID, 3)),
    _const_spec((1, 3)),
    _const_spec((1, _HID)),
    _const_spec((1, 1)),
]

_TC_OUT_SPECS = [
    pl.BlockSpec((_B, _P), lambda i: (i, 0)),
    pl.BlockSpec((_B, _P, 3), lambda i: (i, 0, 0)),
]

_TC_OUT_SHAPE = [
    jax.ShapeDtypeStruct((_R, _P), jnp.float32),
    jax.ShapeDtypeStruct((_R, _P, 3), jnp.float32),
]


def _tc_call(*args):
    return pl.pallas_call(
        _tc_body,
        grid=(_R // _B,),
        in_specs=_TC_IN_SPECS,
        out_specs=_TC_OUT_SPECS,
        out_shape=_TC_OUT_SHAPE,
    )(*args)


def kernel(origins, directions, lengths, grid, frequencies,
           W1, b1, W2, b2, Wc1, bc1, Wc2, bc2, Wd, bd):
    pts = origins[:, None, :] + directions[:, None, :] * lengths[..., None]
    dn = directions / jnp.maximum(
        jnp.linalg.norm(directions, axis=-1, keepdims=True), 1e-12)
    gridf = grid.reshape(-1).astype(jnp.float32)

    pt3 = pts.reshape(_NPT, 3).T
    mask2d = _make_sc_mask()(pt3, gridf).reshape(_R, _P)

    f90 = jnp.tile(frequencies, 3).reshape(1, 3 * _NHARM)
    dens2d, colors = _tc_call(
        pts[..., 0], pts[..., 1], pts[..., 2], dn, mask2d, f90,
        W1[:3 * _NHARM], W1[3 * _NHARM:], b1.reshape(1, _HID),
        W2, b2.reshape(1, _HID),
        Wc1[:_HID], Wc1[_HID:_HID + 3 * _NHARM], Wc1[_HID + 3 * _NHARM:],
        bc1.reshape(1, _HID),
        Wc2, bc2.reshape(1, 3),
        Wd.reshape(1, _HID), bd.reshape(1, 1))
    return dens2d.reshape(_R, _P, 1), colors
